# baseline (speedup 1.0000x reference)
import jax
import jax.numpy as jnp
from jax.experimental import pallas as pl
from jax.experimental.pallas import tpu as pltpu

BM = 256


def _pass_a(x_ref, adj_ref, w1_ref, b1_ref, w2_ref, g_ref, y_scr):
    i = pl.program_id(0)

    @pl.when(i == 0)
    def _():
        y_scr[...] = jnp.dot(x_ref[...], w1_ref[...],
                             preferred_element_type=jnp.float32)

    a = adj_ref[...]
    h = jnp.maximum(
        jnp.dot(a, y_scr[...], preferred_element_type=jnp.float32)
        + b1_ref[...], 0.0)
    g_ref[...] = jnp.dot(h, w2_ref[...], preferred_element_type=jnp.float32)


def _pass_b(adj_ref, g_ref, b2_ref, wfct_ref, bfc_ref, out_ref):
    z = jnp.dot(adj_ref[...], g_ref[...],
                preferred_element_type=jnp.float32) + b2_ref[...]
    o = jnp.dot(z, wfct_ref[...], preferred_element_type=jnp.float32) + bfc_ref[...]
    m = jnp.max(o, axis=1, keepdims=True)
    e = o - m
    out_ref[...] = e - jnp.log(jnp.sum(jnp.exp(e), axis=1, keepdims=True))


def kernel(input, adj, labels, W1, b1, W2, b2, Wfc, bfc):
    x = input
    n, nfeat = x.shape
    nhid = W1.shape[1]
    nclass = W2.shape[1]
    nb = (n + BM - 1) // BM

    b1r = b1.reshape(1, -1)
    b2r = b2.reshape(1, -1)
    bfcr = bfc.reshape(1, -1)
    wfct = Wfc.T

    g = pl.pallas_call(
        _pass_a,
        grid=(nb,),
        in_specs=[
            pl.BlockSpec((n, nfeat), lambda i: (0, 0)),
            pl.BlockSpec((BM, n), lambda i: (i, 0)),
            pl.BlockSpec((nfeat, nhid), lambda i: (0, 0)),
            pl.BlockSpec((1, nhid), lambda i: (0, 0)),
            pl.BlockSpec((nhid, nclass), lambda i: (0, 0)),
        ],
        out_specs=pl.BlockSpec((BM, nclass), lambda i: (i, 0)),
        out_shape=jax.ShapeDtypeStruct((n, nclass), jnp.float32),
        scratch_shapes=[pltpu.VMEM((n, nhid), jnp.float32)],
    )(x, adj, W1, b1r, W2)

    out = pl.pallas_call(
        _pass_b,
        grid=(nb,),
        in_specs=[
            pl.BlockSpec((BM, n), lambda i: (i, 0)),
            pl.BlockSpec((n, nclass), lambda i: (0, 0)),
            pl.BlockSpec((1, nclass), lambda i: (0, 0)),
            pl.BlockSpec((nclass, nclass), lambda i: (0, 0)),
            pl.BlockSpec((1, nclass), lambda i: (0, 0)),
        ],
        out_specs=pl.BlockSpec((BM, nclass), lambda i: (i, 0)),
        out_shape=jax.ShapeDtypeStruct((n, nclass), jnp.float32),
    )(adj, g, b2r, wfct, bfcr)
    return out
